# baseline (speedup 1.0000x reference)
import functools
import math

import jax
import jax.numpy as jnp
from jax import lax
from jax.experimental import pallas as pl
from jax.experimental.pallas import tpu as pltpu
from jax.experimental.pallas import tpu_sc as plsc

BN = 16384
B = 256
V = 100000
D_ATOM = 64
D_T = 64
HALF_T = D_T // 2
D_FF = 128
H = 128

ROWS = 2048
NBLK = BN // ROWS


def _make_sc_gather():
    nc, ns = 2, 16
    nw = nc * ns
    b_per_w = BN // nw
    mesh = plsc.VectorSubcoreMesh(core_axis_name="c", subcore_axis_name="s")

    @functools.partial(
        pl.kernel,
        mesh=mesh,
        out_type=jax.ShapeDtypeStruct((BN, D_ATOM), jnp.float32),
        scratch_types=[
            pltpu.VMEM((b_per_w,), jnp.int32),
            pltpu.VMEM((b_per_w, D_ATOM), jnp.float32),
            pltpu.SemaphoreType.DMA,
        ],
        compiler_params=pltpu.CompilerParams(use_tc_tiling_on_sc=False),
    )
    def gather_k(table_hbm, idx_hbm, out_hbm, idx_v, rows_v, sem):
        wid = lax.axis_index("s") * nc + lax.axis_index("c")
        base = wid * b_per_w
        pltpu.sync_copy(idx_hbm.at[pl.ds(base, b_per_w)], idx_v)
        pltpu.async_copy(table_hbm.at[idx_v], rows_v, sem).wait()
        pltpu.sync_copy(rows_v, out_hbm.at[pl.ds(base, b_per_w)])

    return gather_k


_sc_gather_cache = []


def _sc_gather(table, idx):
    if not _sc_gather_cache:
        _sc_gather_cache.append(_make_sc_gather())
    return _sc_gather_cache[0](table, idx)


def _tc_body(t_ref, bid_ref, c_ref, m_ref, s_ref, e_ref, atom_ref,
             w1_ref, b1_ref, w2_ref, b2_ref, w3_ref, b3_ref, out_ref):
    ji = lax.broadcasted_iota(jnp.int32, (B, D_T), 1)
    j = ji.astype(jnp.float32)
    jh = jnp.where(j >= HALF_T, j - HALF_T, j)
    freqs = jnp.exp(jh * (-math.log(10000.0) / HALF_T))
    args = t_ref[...] * freqs
    temb = jnp.where(j < HALF_T, jnp.sin(args), jnp.cos(args))

    bid = bid_ref[...]
    onehot = (bid == lax.broadcasted_iota(jnp.int32, (ROWS, B), 1)).astype(jnp.float32)
    t_full = jnp.dot(onehot, temb, preferred_element_type=jnp.float32)

    x = jnp.concatenate([c_ref[...], m_ref[...], s_ref[...], e_ref[...]], axis=-1)
    h = jnp.maximum(jnp.dot(x, w1_ref[...], preferred_element_type=jnp.float32) + b1_ref[...], 0.0)
    h = jnp.maximum(jnp.dot(h, w2_ref[...], preferred_element_type=jnp.float32) + b2_ref[...], 0.0)
    ff = jnp.dot(h, w3_ref[...], preferred_element_type=jnp.float32) + b3_ref[...]

    out_ref[...] = jnp.concatenate([atom_ref[...], t_full, ff], axis=-1)


def _tc_call(t2, bid2, charge, mass, sigma, epsilon, atom_emb,
             w1, b1, w2, b2, w3, b3, interpret=False):
    row_spec = lambda w: pl.BlockSpec((ROWS, w), lambda i: (i, 0))
    full = lambda a, b: pl.BlockSpec((a, b), lambda i: (0, 0))
    return pl.pallas_call(
        _tc_body,
        grid=(NBLK,),
        in_specs=[
            full(B, 1),
            row_spec(1),
            row_spec(1), row_spec(1), row_spec(1), row_spec(1),
            row_spec(D_ATOM),
            full(4, H), full(1, H),
            full(H, H), full(1, H),
            full(H, D_FF), full(1, D_FF),
        ],
        out_specs=pl.BlockSpec((ROWS, D_ATOM + D_T + D_FF), lambda i: (i, 0)),
        out_shape=jax.ShapeDtypeStruct((BN, D_ATOM + D_T + D_FF), jnp.float32),
        interpret=interpret,
    )(t2, bid2, charge, mass, sigma, epsilon, atom_emb, w1, b1, w2, b2, w3, b3)


def kernel(atom_type, t_interpolant, batch_ids, charge, mass, sigma, epsilon,
           atom_table, W1, b1, W2, b2, W3, b3):
    atom_emb = _sc_gather(atom_table, atom_type.astype(jnp.int32))
    t2 = t_interpolant.astype(jnp.float32).reshape(B, 1)
    bid2 = batch_ids.astype(jnp.int32).reshape(BN, 1)
    return _tc_call(
        t2, bid2,
        charge.astype(jnp.float32), mass.astype(jnp.float32),
        sigma.astype(jnp.float32), epsilon.astype(jnp.float32),
        atom_emb,
        W1, b1.reshape(1, H), W2, b2.reshape(1, H), W3, b3.reshape(1, D_FF),
    )


# speedup vs baseline: 1.3823x; 1.3823x over previous
import functools
import math

import jax
import jax.numpy as jnp
from jax import lax
from jax.experimental import pallas as pl
from jax.experimental.pallas import tpu as pltpu
from jax.experimental.pallas import tpu_sc as plsc

BN = 16384
B = 256
V = 100000
D_ATOM = 64
D_T = 64
HALF_T = D_T // 2
D_FF = 128
H = 128
D_OUT = D_ATOM + D_T + D_FF

ROWS = 2048
NBLK = BN // ROWS


def _make_sc_gather():
    nc, ns = 2, 16
    nw = nc * ns
    b_per_w = BN // nw
    mesh = plsc.VectorSubcoreMesh(core_axis_name="c", subcore_axis_name="s")

    @functools.partial(
        pl.kernel,
        mesh=mesh,
        out_type=jax.ShapeDtypeStruct((BN, 128), jnp.float32),
        scratch_types=[
            pltpu.VMEM((b_per_w,), jnp.int32),
            pltpu.VMEM((b_per_w, D_ATOM), jnp.float32),
            pltpu.SemaphoreType.DMA,
        ],
        compiler_params=pltpu.CompilerParams(use_tc_tiling_on_sc=False),
    )
    def gather_k(table_hbm, idx_hbm, out_hbm, idx_v, rows_v, sem):
        wid = lax.axis_index("s") * nc + lax.axis_index("c")
        base = wid * b_per_w
        pltpu.sync_copy(idx_hbm.at[pl.ds(base, b_per_w)], idx_v)
        pltpu.async_copy(table_hbm.at[idx_v], rows_v, sem).wait()
        pltpu.sync_copy(rows_v, out_hbm.at[pl.ds(base, b_per_w), pl.ds(0, D_ATOM)])

    return gather_k


_sc_gather_cache = []


def _sc_gather(table, idx):
    if not _sc_gather_cache:
        _sc_gather_cache.append(_make_sc_gather())
    return _sc_gather_cache[0](table, idx)


def _tc_body(t_ref, bid_ref, ffin_ref, atom_ref,
             w1_ref, b1_ref, w2_ref, b2_ref, w3_ref, b3_ref, out_ref):
    ji = lax.broadcasted_iota(jnp.int32, (D_T, B), 0)
    j = ji.astype(jnp.float32)
    jh = jnp.where(j >= HALF_T, j - HALF_T, j)
    freqs = jnp.exp(jh * (-math.log(10000.0) / HALF_T))
    args = t_ref[...] * freqs
    tembT = jnp.where(ji < HALF_T, jnp.sin(args), jnp.cos(args))

    ohT = (lax.broadcasted_iota(jnp.int32, (B, ROWS), 0) == bid_ref[...]).astype(jnp.float32)
    t_full = lax.dot_general(ohT, tembT, (((0,), (1,)), ((), ())),
                             preferred_element_type=jnp.float32)

    x = ffin_ref[...]
    h = jnp.maximum(lax.dot_general(x, w1_ref[...], (((0,), (0,)), ((), ())),
                                    preferred_element_type=jnp.float32) + b1_ref[...], 0.0)
    h = jnp.maximum(jnp.dot(h, w2_ref[...], preferred_element_type=jnp.float32) + b2_ref[...], 0.0)
    ff = jnp.dot(h, w3_ref[...], preferred_element_type=jnp.float32) + b3_ref[...]

    out_ref[...] = jnp.concatenate([atom_ref[:, :D_ATOM], t_full, ff], axis=-1)


def _tc_call(t_row, bid_row, ffin, atom_emb, w1, b1, w2, b2, w3, b3,
             interpret=False):
    return pl.pallas_call(
        _tc_body,
        grid=(NBLK,),
        in_specs=[
            pl.BlockSpec((1, B), lambda i: (0, 0)),
            pl.BlockSpec((1, ROWS), lambda i: (0, i)),
            pl.BlockSpec((4, ROWS), lambda i: (0, i)),
            pl.BlockSpec((ROWS, 128), lambda i: (i, 0)),
            pl.BlockSpec((4, H), lambda i: (0, 0)),
            pl.BlockSpec((1, H), lambda i: (0, 0)),
            pl.BlockSpec((H, H), lambda i: (0, 0)),
            pl.BlockSpec((1, H), lambda i: (0, 0)),
            pl.BlockSpec((H, D_FF), lambda i: (0, 0)),
            pl.BlockSpec((1, D_FF), lambda i: (0, 0)),
        ],
        out_specs=pl.BlockSpec((ROWS, D_OUT), lambda i: (i, 0)),
        out_shape=jax.ShapeDtypeStruct((BN, D_OUT), jnp.float32),
        interpret=interpret,
    )(t_row, bid_row, ffin, atom_emb, w1, b1, w2, b2, w3, b3)


def kernel(atom_type, t_interpolant, batch_ids, charge, mass, sigma, epsilon,
           atom_table, W1, b1, W2, b2, W3, b3):
    atom_emb = _sc_gather(atom_table, atom_type.astype(jnp.int32))
    t_row = t_interpolant.astype(jnp.float32).reshape(1, B)
    bid_row = batch_ids.astype(jnp.int32).reshape(1, BN)
    ffin = jnp.concatenate(
        [charge.astype(jnp.float32).T, mass.astype(jnp.float32).T,
         sigma.astype(jnp.float32).T, epsilon.astype(jnp.float32).T], axis=0)
    return _tc_call(
        t_row, bid_row, ffin, atom_emb,
        W1, b1.reshape(1, H), W2, b2.reshape(1, H), W3, b3.reshape(1, D_FF),
    )
